# baseline (speedup 1.0000x reference)
import functools

import jax
import jax.numpy as jnp
from jax import lax
from jax.experimental import pallas as pl
from jax.experimental.pallas import tpu as pltpu
from jax.experimental.pallas import tpu_sc as plsc

_VOCAB = 100000
_CTX = 2048
_DIM = 1024
_BATCH = 4

_NC = 2
_NS = 16
_NW = _NC * _NS
_PW = _CTX // _NW
_SUB = 32
_NSTEP = _PW // _SUB
_LANES = 16


def _body(x_hbm, tok_hbm, pos_hbm, out_hbm, idx_v, pos_v, rows_v, sem):
    wid = lax.axis_index("s") * _NC + lax.axis_index("c")
    p0 = wid * _PW

    pltpu.sync_copy(pos_hbm.at[pl.ds(p0, _PW)], pos_v)

    for b in range(_BATCH):
        pltpu.sync_copy(x_hbm.at[b, pl.ds(p0, _PW)], idx_v)
        for c in range(_NSTEP):
            pltpu.async_copy(
                tok_hbm.at[idx_v.at[pl.ds(c * _SUB, _SUB)]], rows_v, sem
            ).wait()

            def add_row(r, _, c=c):
                for d in range(_DIM // _LANES):
                    sl = pl.ds(d * _LANES, _LANES)
                    rows_v[r, sl] = rows_v[r, sl] + pos_v[c * _SUB + r, sl]
                return 0

            lax.fori_loop(0, _SUB, add_row, 0)

            pltpu.sync_copy(
                rows_v, out_hbm.at[b, pl.ds(p0 + c * _SUB, _SUB)]
            )


def kernel(x, token_table, pos_table):
    mesh = plsc.VectorSubcoreMesh(core_axis_name="c", subcore_axis_name="s")
    run = functools.partial(
        pl.kernel,
        mesh=mesh,
        out_type=jax.ShapeDtypeStruct((_BATCH, _CTX, _DIM), jnp.float32),
        scratch_types=[
            pltpu.VMEM((_PW,), jnp.int32),
            pltpu.VMEM((_PW, _DIM), jnp.float32),
            pltpu.VMEM((_SUB, _DIM), jnp.float32),
            pltpu.SemaphoreType.DMA,
        ],
    )(_body)
    return run(x, token_table, pos_table)


# speedup vs baseline: 1.1218x; 1.1218x over previous
import functools

import jax
import jax.numpy as jnp
from jax import lax
from jax.experimental import pallas as pl
from jax.experimental.pallas import tpu as pltpu
from jax.experimental.pallas import tpu_sc as plsc

_VOCAB = 100000
_CTX = 2048
_DIM = 1024
_BATCH = 4

_NC = 2
_NS = 16
_NW = _NC * _NS
_PW = _CTX // _NW
_SUB = 16
_NSTEP = _PW // _SUB
_STEPS = _BATCH * _NSTEP
_NBUF = 3
_LANES = 16


def _body(x_hbm, tok_hbm, pos_hbm, out_hbm, idx_v, pos_v,
          rows0, rows1, rows2, gs0, gs1, gs2, os0, os1, os2):
    wid = lax.axis_index("s") * _NC + lax.axis_index("c")
    p0 = wid * _PW

    rows = [rows0, rows1, rows2]
    gsem = [gs0, gs1, gs2]
    osem = [os0, os1, os2]

    for b in range(_BATCH):
        pltpu.sync_copy(x_hbm.at[b, pl.ds(p0, _PW)], idx_v.at[b])
    pltpu.sync_copy(pos_hbm.at[pl.ds(p0, _PW)], pos_v)

    gd = {}
    od = {}

    def gather(s):
        b, c = divmod(s, _NSTEP)
        gd[s] = pltpu.async_copy(
            tok_hbm.at[idx_v.at[b, pl.ds(c * _SUB, _SUB)]],
            rows[s % _NBUF], gsem[s % _NBUF])

    def outcopy(s):
        b, c = divmod(s, _NSTEP)
        od[s] = pltpu.async_copy(
            rows[s % _NBUF],
            out_hbm.at[b, pl.ds(p0 + c * _SUB, _SUB)], osem[s % _NBUF])

    def add_pos(s):
        c = s % _NSTEP
        buf = rows[s % _NBUF]

        def add_row(r, _):
            for d in range(_DIM // _LANES):
                sl = pl.ds(d * _LANES, _LANES)
                plsc.addupdate(buf.at[r, sl], pos_v[c * _SUB + r, sl])
            return 0

        lax.fori_loop(0, _SUB, add_row, 0)

    for s in range(_NBUF):
        gather(s)
    for s in range(_STEPS):
        gd[s].wait()
        if s >= 1 and s - 1 + _NBUF < _STEPS:
            od[s - 1].wait()
            gather(s - 1 + _NBUF)
        add_pos(s)
        outcopy(s)
    for s in range(_STEPS - _NBUF, _STEPS):
        od[s].wait()


def kernel(x, token_table, pos_table):
    mesh = plsc.VectorSubcoreMesh(core_axis_name="c", subcore_axis_name="s")
    run = functools.partial(
        pl.kernel,
        mesh=mesh,
        out_type=jax.ShapeDtypeStruct((_BATCH, _CTX, _DIM), jnp.float32),
        scratch_types=[
            pltpu.VMEM((_BATCH, _PW), jnp.int32),
            pltpu.VMEM((_PW, _DIM), jnp.float32),
            pltpu.VMEM((_SUB, _DIM), jnp.float32),
            pltpu.VMEM((_SUB, _DIM), jnp.float32),
            pltpu.VMEM((_SUB, _DIM), jnp.float32),
            pltpu.SemaphoreType.DMA,
            pltpu.SemaphoreType.DMA,
            pltpu.SemaphoreType.DMA,
            pltpu.SemaphoreType.DMA,
            pltpu.SemaphoreType.DMA,
            pltpu.SemaphoreType.DMA,
        ],
    )(_body)
    return run(x, token_table, pos_table)
